# baseline (speedup 1.0000x reference)
import jax
import jax.numpy as jnp
from jax.experimental import pallas as pl
from jax.experimental.pallas import tpu as pltpu


def kernel(batch_hidden_states, lengths):
    B, T, H = batch_hidden_states.shape

    def body(len_ref, x_hbm, o_hbm, sem):
        copies = []
        for b in range(B):
            idx = jnp.clip(len_ref[b] - 1, 0, T - 1)
            copies.append(
                pltpu.make_async_copy(
                    x_hbm.at[b, pl.ds(idx, 1), :],
                    o_hbm.at[pl.ds(b, 1), :],
                    sem,
                )
            )
        for c in copies:
            c.start()
        for c in copies:
            c.wait()

    return pl.pallas_call(
        body,
        grid_spec=pltpu.PrefetchScalarGridSpec(
            num_scalar_prefetch=1,
            grid=(1,),
            in_specs=[pl.BlockSpec(memory_space=pl.ANY)],
            out_specs=pl.BlockSpec(memory_space=pl.ANY),
            scratch_shapes=[pltpu.SemaphoreType.DMA],
        ),
        out_shape=jax.ShapeDtypeStruct((B, H), jnp.float32),
    )(lengths.astype(jnp.int32), batch_hidden_states)


# speedup vs baseline: 1.6709x; 1.6709x over previous
import jax
import jax.numpy as jnp
from jax.experimental import pallas as pl
from jax.experimental.pallas import tpu as pltpu


def kernel(batch_hidden_states, lengths):
    B, T, H = batch_hidden_states.shape

    def body(len_ref, x_hbm, o_ref, sem):
        copies = []
        for b in range(B):
            idx = jnp.clip(len_ref[b] - 1, 0, T - 1)
            copies.append(
                pltpu.make_async_copy(
                    x_hbm.at[b, pl.ds(idx, 1), :],
                    o_ref.at[pl.ds(b, 1), :],
                    sem,
                )
            )
        for c in copies:
            c.start()
        for c in copies:
            c.wait()

    return pl.pallas_call(
        body,
        grid_spec=pltpu.PrefetchScalarGridSpec(
            num_scalar_prefetch=1,
            grid=(1,),
            in_specs=[pl.BlockSpec(memory_space=pl.ANY)],
            out_specs=pl.BlockSpec((B, H), lambda i, len_ref: (0, 0)),
            scratch_shapes=[pltpu.SemaphoreType.DMA],
        ),
        out_shape=jax.ShapeDtypeStruct((B, H), jnp.float32),
    )(lengths.astype(jnp.int32), batch_hidden_states)


# speedup vs baseline: 1.6816x; 1.0064x over previous
import jax
import jax.numpy as jnp
from jax.experimental import pallas as pl
from jax.experimental.pallas import tpu as pltpu


def kernel(batch_hidden_states, lengths):
    B, T, H = batch_hidden_states.shape

    def body(len_ref, x_hbm, o_ref, sem):
        copies = []
        for b in range(B):
            idx = jnp.clip(len_ref[b] - 1, 0, T - 1)
            copies.append(
                pltpu.make_async_copy(
                    x_hbm.at[b, pl.ds(idx, 1), :],
                    o_ref.at[pl.ds(b, 1), :],
                    sem,
                )
            )
        for c in copies:
            c.start()
        for c in copies:
            c.wait()

    return pl.pallas_call(
        body,
        grid_spec=pltpu.PrefetchScalarGridSpec(
            num_scalar_prefetch=1,
            grid=(),
            in_specs=[pl.BlockSpec(memory_space=pl.ANY)],
            out_specs=pl.BlockSpec((B, H), lambda len_ref: (0, 0)),
            scratch_shapes=[pltpu.SemaphoreType.DMA],
        ),
        out_shape=jax.ShapeDtypeStruct((B, H), jnp.float32),
    )(lengths.astype(jnp.int32), batch_hidden_states)
